# Initial kernel scaffold; baseline (speedup 1.0000x reference)
import jax
import jax.numpy as jnp
from jax.experimental import pallas as pl


def kernel(seq, embed, W1, b1, W2, b2, gamma, beta, Wq, bq, Wout, bout):
    raise NotImplementedError("write your pallas kernel here")


# speedup vs baseline: 113.6940x; 113.6940x over previous
import functools

import jax
import jax.numpy as jnp
from jax import lax
from jax.experimental import pallas as pl
from jax.experimental.pallas import tpu as pltpu
from jax.experimental.pallas import tpu_sc as plsc

_H = 64
_V = 64
_K = 6


def _make_sc_hist(B, L):
    info = plsc.get_sparse_core_info()
    NC, NS, NL = info.num_cores, info.num_subcores, info.num_lanes
    NW = NC * NS
    assert B % NW == 0 and L % NL == 0
    rows_per_w = B // NW
    groups = L // NL

    mesh = plsc.VectorSubcoreMesh(core_axis_name="c", subcore_axis_name="s")

    @functools.partial(
        pl.kernel,
        out_type=jax.ShapeDtypeStruct((B, _V), jnp.int32),
        mesh=mesh,
        compiler_params=pltpu.CompilerParams(needs_layout_passes=False),
        scratch_types=[
            pltpu.VMEM((rows_per_w, L), jnp.int32),
            pltpu.VMEM((rows_per_w * NL * _V,), jnp.int32),
            pltpu.VMEM((rows_per_w, _V), jnp.int32),
        ],
    )
    def sc_hist(seq_hbm, out_hbm, seq_v, hist_v, rows_v):
        wid = lax.axis_index("s") * NC + lax.axis_index("c")
        base = wid * rows_per_w
        pltpu.sync_copy(seq_hbm.at[pl.ds(base, rows_per_w)], seq_v)

        lanebase = lax.iota(jnp.int32, NL) * _V
        ones = jnp.ones((NL,), jnp.int32)
        zeros = jnp.zeros((NL,), jnp.int32)

        def zero_body(i, carry):
            hist_v[pl.ds(i * NL, NL)] = zeros
            return carry

        lax.fori_loop(0, rows_per_w * NL * _V // NL, zero_body, 0)

        for r in range(rows_per_w):
            hbase = r * NL * _V

            def grp_body(g, carry, r=r, hbase=hbase):
                v = seq_v[r, pl.ds(g * NL, NL)]
                plsc.addupdate_scatter(hist_v, [hbase + lanebase + v], ones)
                return carry

            lax.fori_loop(0, groups, grp_body, 0)

        for r in range(rows_per_w):
            hbase = r * NL * _V
            for j in range(_V // NL):
                acc = hist_v[pl.ds(hbase + j * NL, NL)]
                for lane_i in range(1, NL):
                    acc = acc + hist_v[pl.ds(hbase + lane_i * _V + j * NL, NL)]
                rows_v[r, pl.ds(j * NL, NL)] = acc

        pltpu.sync_copy(rows_v, out_hbm.at[pl.ds(base, rows_per_w)])

    return sc_hist


def _dense_body(counts_ref, tail_ref, embed_ref, W1_ref, b1_ref, W2_ref,
                b2_ref, gamma_ref, beta_ref, Wq_ref, bq_ref, Wout_ref,
                bout_ref, out_ref):
    f32 = jnp.float32
    mm = lambda a, b: lax.dot_general(a, b, (((1,), (0,)), ((), ())),
                                      preferred_element_type=f32)
    e = embed_ref[...]
    a1 = jnp.maximum(mm(e, W1_ref[...]) + b1_ref[...], 0.0)
    ff = mm(a1, W2_ref[...]) + b2_ref[...]
    x = e + ff
    mu = jnp.mean(x, axis=-1, keepdims=True)
    var = jnp.mean((x - mu) ** 2, axis=-1, keepdims=True)
    h = (x - mu) / jnp.sqrt(var + 1e-5) * gamma_ref[...] + beta_ref[...]
    h_t = jnp.transpose(h)

    n2_col = jnp.sum(h * h, axis=-1, keepdims=True)
    n2_row = jnp.sum(h_t * h_t, axis=0, keepdims=True)
    G = (n2_col > n2_row).astype(f32)

    iv = lax.broadcasted_iota(jnp.int32, (1, _V), 1)
    oh0 = (tail_ref[:, 0:1] == iv).astype(f32)
    oh1 = (tail_ref[:, 1:2] == iv).astype(f32)
    oh2 = (tail_ref[:, 2:3] == iv).astype(f32)
    cf = counts_ref[...].astype(f32) - oh0 - oh1 - oh2

    C = mm(cf, G)
    m = jnp.minimum(jnp.maximum(float(_K) - C, 0.0), cf)

    h_last = mm(oh2, h)
    q = mm(h_last, Wq_ref[...]) + bq_ref[...]
    logits = mm(q, h_t) * (1.0 / (_H ** 0.5))

    lm = jnp.where(m > 0.0, logits, -1e30)
    mx = jnp.max(lm, axis=-1, keepdims=True)
    p = m * jnp.exp(lm - mx)
    w = p / jnp.sum(p, axis=-1, keepdims=True)
    ctx = mm(w, h)
    out_ref[...] = mm(ctx, Wout_ref[...]) + bout_ref[...]


def kernel(seq, embed, W1, b1, W2, b2, gamma, beta, Wq, bq, Wout, bout):
    B, L = seq.shape
    counts = _make_sc_hist(B, L)(seq)
    tail = lax.slice(seq, (0, L - 3), (B, L))
    row = lambda v: v.reshape(1, -1)
    out = pl.pallas_call(
        _dense_body,
        out_shape=jax.ShapeDtypeStruct((B, _V), jnp.float32),
    )(counts, tail, embed, W1, row(b1), W2, row(b2), row(gamma), row(beta),
      Wq, row(bq), Wout, row(bout))
    return out


# speedup vs baseline: 156.7488x; 1.3787x over previous
import functools

import jax
import jax.numpy as jnp
from jax import lax
from jax.experimental import pallas as pl
from jax.experimental.pallas import tpu as pltpu
from jax.experimental.pallas import tpu_sc as plsc

_H = 64
_V = 64
_K = 6


def _make_sc_hist(B, L):
    info = plsc.get_sparse_core_info()
    NC, NS, NL = info.num_cores, info.num_subcores, info.num_lanes
    NW = NC * NS
    assert B % NW == 0 and L % NL == 0
    rows_per_w = B // NW
    groups = L // NL

    mesh = plsc.VectorSubcoreMesh(core_axis_name="c", subcore_axis_name="s")

    @functools.partial(
        pl.kernel,
        out_type=jax.ShapeDtypeStruct((B, _V), jnp.int32),
        mesh=mesh,
        compiler_params=pltpu.CompilerParams(needs_layout_passes=False),
        scratch_types=[
            pltpu.VMEM((rows_per_w, L), jnp.int32),
            pltpu.VMEM((rows_per_w * NL * _V,), jnp.int32),
            pltpu.VMEM((rows_per_w, _V), jnp.int32),
        ],
    )
    def sc_hist(seq_hbm, out_hbm, seq_v, hist_v, rows_v):
        wid = lax.axis_index("s") * NC + lax.axis_index("c")
        base = wid * rows_per_w
        pltpu.sync_copy(seq_hbm.at[pl.ds(base, rows_per_w)], seq_v)

        lanebase = lax.iota(jnp.int32, NL) * _V
        ones = jnp.ones((NL,), jnp.int32)
        zeros = jnp.zeros((NL,), jnp.int32)

        def zero_body(i, carry):
            hist_v[pl.ds(i * NL, NL)] = zeros
            return carry

        lax.fori_loop(0, rows_per_w * NL * _V // NL, zero_body, 0)

        for r in range(rows_per_w):
            hbase = r * NL * _V

            @plsc.parallel_loop(0, groups, 1, unroll=16)
            def grp_body(g, r=r, hbase=hbase):
                v = seq_v[r, pl.ds(g * NL, NL)]
                plsc.addupdate_scatter(hist_v, [hbase + lanebase + v], ones)

        for r in range(rows_per_w):
            hbase = r * NL * _V
            for j in range(_V // NL):
                acc = hist_v[pl.ds(hbase + j * NL, NL)]
                for lane_i in range(1, NL):
                    acc = acc + hist_v[pl.ds(hbase + lane_i * _V + j * NL, NL)]
                rows_v[r, pl.ds(j * NL, NL)] = acc

        pltpu.sync_copy(rows_v, out_hbm.at[pl.ds(base, rows_per_w)])

    return sc_hist


def _dense_body(counts_ref, tail_ref, embed_ref, W1_ref, b1_ref, W2_ref,
                b2_ref, gamma_ref, beta_ref, Wq_ref, bq_ref, Wout_ref,
                bout_ref, out_ref):
    f32 = jnp.float32
    mm = lambda a, b: lax.dot_general(a, b, (((1,), (0,)), ((), ())),
                                      preferred_element_type=f32)
    e = embed_ref[...]
    a1 = jnp.maximum(mm(e, W1_ref[...]) + b1_ref[...], 0.0)
    ff = mm(a1, W2_ref[...]) + b2_ref[...]
    x = e + ff
    mu = jnp.mean(x, axis=-1, keepdims=True)
    var = jnp.mean((x - mu) ** 2, axis=-1, keepdims=True)
    h = (x - mu) / jnp.sqrt(var + 1e-5) * gamma_ref[...] + beta_ref[...]
    h_t = jnp.transpose(h)

    n2_col = jnp.sum(h * h, axis=-1, keepdims=True)
    n2_row = jnp.sum(h_t * h_t, axis=0, keepdims=True)
    G = (n2_col > n2_row).astype(f32)

    iv = lax.broadcasted_iota(jnp.int32, (1, _V), 1)
    oh0 = (tail_ref[:, 0:1] == iv).astype(f32)
    oh1 = (tail_ref[:, 1:2] == iv).astype(f32)
    oh2 = (tail_ref[:, 2:3] == iv).astype(f32)
    cf = counts_ref[...].astype(f32) - oh0 - oh1 - oh2

    C = mm(cf, G)
    m = jnp.minimum(jnp.maximum(float(_K) - C, 0.0), cf)

    h_last = mm(oh2, h)
    q = mm(h_last, Wq_ref[...]) + bq_ref[...]
    logits = mm(q, h_t) * (1.0 / (_H ** 0.5))

    lm = jnp.where(m > 0.0, logits, -1e30)
    mx = jnp.max(lm, axis=-1, keepdims=True)
    p = m * jnp.exp(lm - mx)
    w = p / jnp.sum(p, axis=-1, keepdims=True)
    ctx = mm(w, h)
    out_ref[...] = mm(ctx, Wout_ref[...]) + bout_ref[...]


def kernel(seq, embed, W1, b1, W2, b2, gamma, beta, Wq, bq, Wout, bout):
    B, L = seq.shape
    counts = _make_sc_hist(B, L)(seq)
    tail = lax.slice(seq, (0, L - 3), (B, L))
    row = lambda v: v.reshape(1, -1)
    out = pl.pallas_call(
        _dense_body,
        out_shape=jax.ShapeDtypeStruct((B, _V), jnp.float32),
    )(counts, tail, embed, W1, row(b1), W2, row(b2), row(gamma), row(beta),
      Wq, row(bq), Wout, row(bout))
    return out
